# Initial kernel scaffold; baseline (speedup 1.0000x reference)
import jax
import jax.numpy as jnp
from jax.experimental import pallas as pl


def kernel(x, edge_index, batch, w1_l, b1_l, w1_r, w2_l, b2_l, w2_r, w3_l, b3_l, w3_r, conv_w, conv_b, lin1_w, lin1_b, lin2_w, lin2_b):
    raise NotImplementedError("write your pallas kernel here")


# speedup vs baseline: 3.7036x; 3.7036x over previous
import jax
import jax.numpy as jnp
import numpy as np
from jax import lax
from jax.experimental import pallas as pl
from jax.experimental.pallas import tpu as pltpu
from jax.experimental.pallas import tpu_sc as plsc

N = 10000
NP = 10112
D = 128
H = 64
B = 64
K = 30
E = 320000
CH = 128
CPT = 79
EPT = CH * CPT
EP = EPT * 32
RPT = NP // 16
GP = 2048

_F32 = jnp.float32


def _dot(a, b):
    return lax.dot_general(a, b, (((1,), (0,)), ((), ())),
                           preferred_element_type=_F32)


_SC_CACHE = {}


def _mesh():
    if "mesh" not in _SC_CACHE:
        _SC_CACHE["mesh"] = plsc.VectorSubcoreMesh(
            core_axis_name="c", subcore_axis_name="s")
    return _SC_CACHE["mesh"]


def _floop(n, body_fn):
    lax.fori_loop(jnp.int32(0), jnp.int32(n),
                  lambda i, c: (body_fn(i), c)[1], jnp.int32(0))


def _make_segsum_body(w, with_cnt):
    def body(table, srch, dsth, *rest):
        if with_cnt:
            (acc_out, cnt_out, acc_sh, cnt_sh, cobuf, cocnt, ones_v,
             idx_s, idx_d, rows, sem) = rest
        else:
            acc_out, acc_sh, cobuf, idx_s, idx_d, rows, sem = rest
        c = lax.axis_index("c")
        s = lax.axis_index("s")
        row0 = s * jnp.int32(RPT)
        z16 = jnp.zeros((16,), _F32)

        def _zero(r):
            for j in range(w // 16):
                cobuf[r, pl.ds(j * 16, 16)] = z16
            if with_cnt:
                cocnt[r, :] = z16

        _floop(RPT, _zero)
        pltpu.sync_copy(cobuf, acc_sh.at[pl.ds(row0, RPT)])
        if with_cnt:
            o16 = jnp.ones((16,), _F32)

            def _ones(r):
                ones_v[r, :] = o16

            _floop(CH, _ones)
            pltpu.sync_copy(cocnt, cnt_sh.at[pl.ds(row0, RPT)])
        plsc.subcore_barrier()

        ebase = c * jnp.int32(16 * EPT) + s * jnp.int32(EPT)

        def _edge(k):
            b = ebase + k * jnp.int32(CH)
            pltpu.sync_copy(srch.at[pl.ds(b, CH)], idx_s)
            pltpu.sync_copy(dsth.at[pl.ds(b, CH)], idx_d)
            pltpu.async_copy(table.at[idx_s], rows, sem).wait()
            pltpu.sync_copy(rows, acc_sh.at[idx_d], add=True)
            if with_cnt:
                pltpu.sync_copy(ones_v, cnt_sh.at[idx_d], add=True)

        _floop(CPT, _edge)
        plsc.subcore_barrier()
        pltpu.sync_copy(acc_sh.at[pl.ds(row0, RPT)], cobuf)
        pltpu.sync_copy(cobuf, acc_out.at[c, pl.ds(row0, RPT)])
        if with_cnt:
            pltpu.sync_copy(cnt_sh.at[pl.ds(row0, RPT)], cocnt)
            pltpu.sync_copy(cocnt, cnt_out.at[c, pl.ds(row0, RPT)])

    return body


def _segsum(table, src, dst, w, with_cnt):
    key = ("segsum", w, with_cnt)
    if key not in _SC_CACHE:
        outs = [jax.ShapeDtypeStruct((2, NP, w), _F32)]
        scratch = [pltpu.VMEM_SHARED((NP, w), _F32)]
        if with_cnt:
            outs.append(jax.ShapeDtypeStruct((2, NP, 16), _F32))
            scratch.append(pltpu.VMEM_SHARED((NP, 16), _F32))
        scratch.append(pltpu.VMEM((RPT, w), _F32))
        if with_cnt:
            scratch.append(pltpu.VMEM((RPT, 16), _F32))
            scratch.append(pltpu.VMEM((CH, 16), _F32))
        scratch += [pltpu.VMEM((CH,), jnp.int32),
                    pltpu.VMEM((CH,), jnp.int32),
                    pltpu.VMEM((CH, w), _F32),
                    pltpu.SemaphoreType.DMA]
        _SC_CACHE[key] = pl.kernel(
            _make_segsum_body(w, with_cnt),
            out_type=tuple(outs) if with_cnt else outs[0],
            mesh=_mesh(),
            compiler_params=pltpu.CompilerParams(use_tc_tiling_on_sc=False),
            scratch_types=scratch,
        )
    return _SC_CACHE[key](table, src, dst)


def _gather_body(h3, idxh, out, idxv, rows, sem):
    wid = lax.axis_index("c") * jnp.int32(16) + lax.axis_index("s")
    base = wid * jnp.int32(GP // 32)
    pltpu.sync_copy(idxh.at[pl.ds(base, GP // 32)], idxv)
    pltpu.async_copy(h3.at[idxv], rows, sem).wait()
    pltpu.sync_copy(rows, out.at[pl.ds(base, GP // 32)])


def _gather_rows(h3, idx_flat):
    if "gather" not in _SC_CACHE:
        _SC_CACHE["gather"] = pl.kernel(
            _gather_body,
            out_type=jax.ShapeDtypeStruct((GP, H), _F32),
            mesh=_mesh(),
            compiler_params=pltpu.CompilerParams(use_tc_tiling_on_sc=False),
            scratch_types=[
                pltpu.VMEM((GP // 32,), jnp.int32),
                pltpu.VMEM((GP // 32, H), _F32),
                pltpu.SemaphoreType.DMA,
            ],
        )
    return _SC_CACHE["gather"](h3, idx_flat)


def _topk_body(h_ref, b_ref, idx_ref, mat_ref):
    biota = lax.broadcasted_iota(jnp.int32, (NP, B), 1)
    riota = lax.broadcasted_iota(jnp.int32, (NP, B), 0)
    last = h_ref[...][:, H - 1:H]
    neg = jnp.float32(-jnp.inf)
    mat_ref[...] = jnp.where(b_ref[...] == biota,
                             jnp.broadcast_to(last, (NP, B)), neg)

    def step(k, carry):
        mat = mat_ref[...]
        m = jnp.max(mat, axis=0, keepdims=True)
        cand = jnp.where(mat == m, riota, jnp.int32(1 << 30))
        w = jnp.min(cand, axis=0, keepdims=True)
        idx_ref[pl.ds(k, 1), :] = jnp.where(m >= 0.0, w, jnp.int32(NP - 1))
        mat_ref[...] = jnp.where(riota == w, neg, mat)
        return carry

    lax.fori_loop(jnp.int32(0), jnp.int32(K), step, jnp.int32(0))


def _topk(h3, batch2d):
    return pl.pallas_call(
        _topk_body,
        out_shape=jax.ShapeDtypeStruct((K, B), jnp.int32),
        scratch_shapes=[pltpu.VMEM((NP, B), _F32)],
    )(h3, batch2d)


def _head_body(g_ref, wb_ref, cb_ref, l1_ref, b1_ref, l2_ref, b2_ref, o_ref):
    y = jnp.maximum(_dot(g_ref[...], wb_ref[...]) + cb_ref[...], 0.0)
    z = jnp.maximum(_dot(y, l1_ref[...]) + b1_ref[...], 0.0)
    o_ref[...] = _dot(z, l2_ref[...]) + b2_ref[...]


def _head(gflat, wbig, cbt, l1t, b1, l2t, b2):
    return pl.pallas_call(
        _head_body,
        out_shape=jax.ShapeDtypeStruct((B, 1), _F32),
    )(gflat, wbig, cbt, l1t, b1, l2t, b2)


def kernel(x, edge_index, batch, w1_l, b1_l, w1_r, w2_l, b2_l, w2_r,
           w3_l, b3_l, w3_r, conv_w, conv_b, lin1_w, lin1_b, lin2_w, lin2_b):
    x = x.astype(_F32)
    ei = edge_index.astype(jnp.int32)
    pad_e = jnp.full((EP - E,), N, jnp.int32)
    src = jnp.concatenate([ei[0], pad_e])
    dst = jnp.concatenate([ei[1], pad_e])
    batch_p = jnp.concatenate(
        [batch.astype(jnp.int32), jnp.full((NP - N,), B, jnp.int32)]
    ).reshape(NP, 1)

    def seg(hp, with_cnt):
        return _segsum(hp, src, dst, H, with_cnt)

    def sage(h, wl, bl, wr, cnt_m, out_cnt=False):
        hp = jnp.pad(h, ((0, NP - h.shape[0]), (0, 0)))
        parts = []
        for j in range(hp.shape[1] // H):
            res = seg(hp[:, j * H:(j + 1) * H], out_cnt and j == 0)
            if out_cnt and j == 0:
                acc, cnt = res
                cnt_m = jnp.maximum((cnt[0] + cnt[1])[:N, 0:1], 1.0)
            else:
                acc = res
            parts.append((acc[0] + acc[1])[:N])
        s = jnp.concatenate(parts, axis=1) if len(parts) > 1 else parts[0]
        mean = s / cnt_m
        out = jax.nn.relu(mean @ wl.T + bl + h @ wr.T)
        return out, cnt_m

    h1, cnt_m = sage(x, w1_l, b1_l, w1_r, None, out_cnt=True)
    h2, _ = sage(h1, w2_l, b2_l, w2_r, cnt_m)
    h3, _ = sage(h2, w3_l, b3_l, w3_r, cnt_m)

    h3p = jnp.pad(h3, ((0, NP - N), (0, 0)))
    idxk = _topk(h3p, batch_p)
    idx_flat = jnp.concatenate(
        [idxk.T.reshape(B * K), jnp.full((GP - B * K,), NP - 1, jnp.int32)])
    g = _gather_rows(h3p, idx_flat)
    gflat = g[:B * K].reshape(B, K * H)

    wb = jnp.zeros((K, H, K - 2, 32), _F32)
    t_ar = jnp.arange(K - 2)
    for dt in range(3):
        wb = wb.at[t_ar + dt, :, t_ar, :].set(
            jnp.broadcast_to(conv_w[:, :, dt].T.astype(_F32), (K - 2, H, 32)))
    wbig = wb.reshape(K * H, (K - 2) * 32)
    cbt = jnp.tile(conv_b.astype(_F32), K - 2).reshape(1, (K - 2) * 32)
    l1t = (lin1_w.reshape(H, 32, K - 2).transpose(0, 2, 1)
           .reshape(H, (K - 2) * 32)).T.astype(_F32)
    b1h = lin1_b.reshape(1, H).astype(_F32)
    l2t = lin2_w.T.astype(_F32)
    b2h = lin2_b.reshape(1, 1).astype(_F32)

    return _head(gflat, wbig, cbt, l1t, b1h, l2t, b2h)
